# baseline (speedup 1.0000x reference)
import functools

import jax
import jax.numpy as jnp
from jax.experimental import pallas as pl
from jax.experimental.pallas import tpu as pltpu


def _rpn_body(x_ref, wk_ref, whead_ref, bhead_ref, bconv_ref,
              cls_ref, loc_ref, xm0_ref, xm1_ref, v_ref,
              *, n, w, margin, n_cls):
    c = x_ref.shape[1]
    next_ = n + 2 * margin

    zl = jnp.zeros((c, margin), jnp.bfloat16)
    pcol = jax.lax.broadcasted_iota(jnp.int32, (1, n), 1) % w
    m_left = pcol != 0
    m_right = pcol != w - 1

    for bi, xm_ref in ((0, xm0_ref), (1, xm1_ref)):
        xm_ref[:, 0:margin] = zl
        xm_ref[:, margin + n:next_] = zl
        xm_ref[:, margin:margin + n] = x_ref[bi].astype(jnp.bfloat16)

        for k in range(9):
            dh, dw = divmod(k, 3)
            s = margin + (dh - 1) * w + (dw - 1)
            blk = xm_ref[:, s:s + n]
            if dw == 0:
                blk = jnp.where(m_left, blk, jnp.bfloat16(0))
            elif dw == 2:
                blk = jnp.where(m_right, blk, jnp.bfloat16(0))
            v_ref[k * c:(k + 1) * c, :] = blk

        y1 = jax.lax.dot_general(
            wk_ref[...], v_ref[...],
            dimension_numbers=(((1,), (0,)), ((), ())),
            preferred_element_type=jnp.float32)
        y1 = jnp.maximum(y1 + bconv_ref[...], 0.0)
        y1 = y1.astype(jnp.bfloat16)

        head = jax.lax.dot_general(
            whead_ref[...], y1, dimension_numbers=(((1,), (0,)), ((), ())),
            preferred_element_type=jnp.float32) + bhead_ref[...]

        half = n_cls // 2
        a = head[0:half, :]
        b = head[half:n_cls, :]
        m = jnp.maximum(a, b)
        ea = jnp.exp(a - m)
        eb = jnp.exp(b - m)
        denom = ea + eb
        cls_ref[bi] = jnp.concatenate([ea / denom, eb / denom], axis=0)
        loc_ref[bi] = head[n_cls:, :]


def kernel(feats, gt_boxes, im_info, W_conv, b_conv, W_cls, b_cls, W_loc, b_loc):
    B, C, H, W = feats.shape
    N = H * W
    M = W + 1
    n_cls = W_cls.shape[0]
    n_loc = W_loc.shape[0]
    n_hd = n_cls + n_loc

    x = feats.reshape(B, C, N)

    wk = W_conv.transpose(0, 2, 3, 1).reshape(C, 9 * C).astype(jnp.bfloat16)
    whead = jnp.concatenate(
        [W_cls.reshape(n_cls, C), W_loc.reshape(n_loc, C)],
        axis=0).astype(jnp.bfloat16)
    bhead = jnp.concatenate([b_cls, b_loc]).reshape(n_hd, 1)

    body = functools.partial(_rpn_body, n=N, w=W, margin=M, n_cls=n_cls)
    cls_flat, loc_flat = pl.pallas_call(
        body,
        grid=(B // 2,),
        in_specs=[
            pl.BlockSpec((2, C, N), lambda b: (b, 0, 0)),
            pl.BlockSpec((C, 9 * C), lambda b: (0, 0)),
            pl.BlockSpec((n_hd, C), lambda b: (0, 0)),
            pl.BlockSpec((n_hd, 1), lambda b: (0, 0)),
            pl.BlockSpec((C, 1), lambda b: (0, 0)),
        ],
        out_specs=[
            pl.BlockSpec((2, n_cls, N), lambda b: (b, 0, 0)),
            pl.BlockSpec((2, n_loc, N), lambda b: (b, 0, 0)),
        ],
        out_shape=[
            jax.ShapeDtypeStruct((B, n_cls, N), jnp.float32),
            jax.ShapeDtypeStruct((B, n_loc, N), jnp.float32),
        ],
        scratch_shapes=[
            pltpu.VMEM((C, N + 2 * M), jnp.bfloat16),
            pltpu.VMEM((C, N + 2 * M), jnp.bfloat16),
            pltpu.VMEM((9 * C, N), jnp.bfloat16),
        ],
        compiler_params=pltpu.CompilerParams(
            dimension_semantics=("arbitrary",)),
    )(x, wk, whead, bhead, b_conv.reshape(C, 1))

    return (cls_flat.reshape(B, n_cls, H, W), loc_flat.reshape(B, n_loc, H, W))


# speedup vs baseline: 1.0030x; 1.0030x over previous
import functools

import jax
import jax.numpy as jnp
from jax.experimental import pallas as pl
from jax.experimental.pallas import tpu as pltpu


def _rpn_body(x_ref, wk_ref, whead_ref, bhead_ref, bconv_ref,
              cls_ref, loc_ref, xm_ref, v_ref, *, n, w, margin, n_cls):
    c = x_ref.shape[1]
    next_ = n + 2 * margin

    zl = jnp.zeros((c, margin), jnp.bfloat16)
    xm_ref[:, 0:margin] = zl
    xm_ref[:, margin + n:next_] = zl
    xm_ref[:, margin:margin + n] = x_ref[0].astype(jnp.bfloat16)

    pcol = jax.lax.broadcasted_iota(jnp.int32, (1, n), 1) % w
    m_left = pcol != 0
    m_right = pcol != w - 1
    for k in range(9):
        dh, dw = divmod(k, 3)
        s = margin + (dh - 1) * w + (dw - 1)
        blk = xm_ref[:, s:s + n]
        if dw == 0:
            blk = jnp.where(m_left, blk, jnp.bfloat16(0))
        elif dw == 2:
            blk = jnp.where(m_right, blk, jnp.bfloat16(0))
        v_ref[k * c:(k + 1) * c, :] = blk

    y1 = jax.lax.dot_general(
        wk_ref[...], v_ref[...],
        dimension_numbers=(((1,), (0,)), ((), ())),
        preferred_element_type=jnp.float32)
    y1 = jnp.maximum(y1 + bconv_ref[...], 0.0)
    y1 = y1.astype(jnp.bfloat16)

    head = jax.lax.dot_general(
        whead_ref[...], y1, dimension_numbers=(((1,), (0,)), ((), ())),
        preferred_element_type=jnp.float32) + bhead_ref[...]

    half = n_cls // 2
    a = head[0:half, :]
    b = head[half:n_cls, :]
    m = jnp.maximum(a, b)
    ea = jnp.exp(a - m)
    eb = jnp.exp(b - m)
    denom = ea + eb
    cls_ref[0] = jnp.concatenate([ea / denom, eb / denom], axis=0)
    loc_ref[0] = head[n_cls:, :]


def kernel(feats, gt_boxes, im_info, W_conv, b_conv, W_cls, b_cls, W_loc, b_loc):
    B, C, H, W = feats.shape
    N = H * W
    M = W + 1
    n_cls = W_cls.shape[0]
    n_loc = W_loc.shape[0]
    n_hd = n_cls + n_loc

    x = feats.reshape(B, C, N)

    wk = W_conv.transpose(0, 2, 3, 1).reshape(C, 9 * C).astype(jnp.bfloat16)
    whead = jnp.concatenate(
        [W_cls.reshape(n_cls, C), W_loc.reshape(n_loc, C)],
        axis=0).astype(jnp.bfloat16)
    bhead = jnp.concatenate([b_cls, b_loc]).reshape(n_hd, 1)

    body = functools.partial(_rpn_body, n=N, w=W, margin=M, n_cls=n_cls)
    cls_flat, loc_flat = pl.pallas_call(
        body,
        grid=(B,),
        in_specs=[
            pl.BlockSpec((1, C, N), lambda b: (b, 0, 0)),
            pl.BlockSpec((C, 9 * C), lambda b: (0, 0)),
            pl.BlockSpec((n_hd, C), lambda b: (0, 0)),
            pl.BlockSpec((n_hd, 1), lambda b: (0, 0)),
            pl.BlockSpec((C, 1), lambda b: (0, 0)),
        ],
        out_specs=[
            pl.BlockSpec((1, n_cls, N), lambda b: (b, 0, 0)),
            pl.BlockSpec((1, n_loc, N), lambda b: (b, 0, 0)),
        ],
        out_shape=[
            jax.ShapeDtypeStruct((B, n_cls, N), jnp.float32),
            jax.ShapeDtypeStruct((B, n_loc, N), jnp.float32),
        ],
        scratch_shapes=[
            pltpu.VMEM((C, N + 2 * M), jnp.bfloat16),
            pltpu.VMEM((9 * C, N), jnp.bfloat16),
        ],
        compiler_params=pltpu.CompilerParams(
            dimension_semantics=("arbitrary",)),
    )(x, wk, whead, bhead, b_conv.reshape(C, 1))

    return (cls_flat.reshape(B, n_cls, H, W), loc_flat.reshape(B, n_loc, H, W))
